# Initial kernel scaffold; baseline (speedup 1.0000x reference)
import jax
import jax.numpy as jnp
from jax.experimental import pallas as pl


def kernel(pos_enc_1D, pos):
    raise NotImplementedError("write your pallas kernel here")


# speedup vs baseline: 3.5820x; 3.5820x over previous
import functools

import jax
import jax.numpy as jnp
from jax import lax
from jax.experimental import pallas as pl
from jax.experimental.pallas import tpu as pltpu
from jax.experimental.pallas import tpu_sc as plsc

D = 128
NC = 2
NS = 16
NW = NC * NS


def kernel(pos_enc_1D, pos):
    B, S = pos.shape
    N = B * S
    per_w = N // NW
    CH = 512
    n_ch = per_w // CH

    idx_flat = pos.reshape(N)
    mesh = plsc.VectorSubcoreMesh(core_axis_name="c", subcore_axis_name="s")

    @functools.partial(
        pl.kernel,
        mesh=mesh,
        out_type=jax.ShapeDtypeStruct((N, D), jnp.float32),
        scratch_types=[
            pltpu.VMEM((per_w,), jnp.int32),
            pltpu.VMEM((CH, D), jnp.float32),
            pltpu.SemaphoreType.DMA,
        ],
    )
    def gather_kernel(table_hbm, idx_hbm, out_hbm, idx_v, rows_v, sem):
        wid = lax.axis_index("s") * NC + lax.axis_index("c")
        base = wid * per_w
        pltpu.sync_copy(idx_hbm.at[pl.ds(base, per_w)], idx_v)

        def body(i, carry):
            pltpu.async_copy(
                table_hbm.at[idx_v.at[pl.ds(i * CH, CH)]], rows_v, sem
            ).wait()
            pltpu.sync_copy(rows_v, out_hbm.at[pl.ds(base + i * CH, CH)])
            return carry

        lax.fori_loop(0, n_ch, body, 0)

    out = gather_kernel(pos_enc_1D, idx_flat)
    return out.reshape(B, S, D)


# speedup vs baseline: 15.4890x; 4.3242x over previous
import functools

import jax
import jax.numpy as jnp
from jax import lax
from jax.experimental import pallas as pl
from jax.experimental.pallas import tpu as pltpu
from jax.experimental.pallas import tpu_sc as plsc

D = 128
NC = 2
NS = 16
NW = NC * NS


def kernel(pos_enc_1D, pos):
    B, S = pos.shape
    V = pos_enc_1D.shape[0]
    N = B * S
    per_w = N // NW
    CH = 400
    n_ch = per_w // CH

    idx_flat = pos.reshape(N)
    mesh = plsc.VectorSubcoreMesh(core_axis_name="c", subcore_axis_name="s")

    @functools.partial(
        pl.kernel,
        mesh=mesh,
        out_type=jax.ShapeDtypeStruct((N, D), jnp.float32),
        scratch_types=[
            pltpu.VMEM((per_w,), jnp.int32),
            pltpu.VMEM((CH, D), jnp.float32),
            pltpu.VMEM((CH, D), jnp.float32),
            pltpu.VMEM_SHARED((V, D), jnp.float32),
            pltpu.SemaphoreType.DMA,
            pltpu.SemaphoreType.DMA,
            pltpu.SemaphoreType.DMA,
        ],
    )
    def gather_kernel(table_hbm, idx_hbm, out_hbm, idx_v, rows0, rows1,
                      table_sp, sem_i, sem_g0, sem_g1):
        cid = lax.axis_index("c")
        sid = lax.axis_index("s")
        wid = sid * NC + cid
        base = wid * per_w

        idx_cp = pltpu.make_async_copy(
            idx_hbm.at[pl.ds(base, per_w)], idx_v, sem_i)
        idx_cp.start()

        @pl.when(sid == 0)
        def _():
            pltpu.sync_copy(table_hbm, table_sp)

        plsc.subcore_barrier()
        idx_cp.wait()

        def start_gather(i, rows, sem):
            pltpu.make_async_copy(
                table_sp.at[idx_v.at[pl.ds(i * CH, CH)]], rows, sem).start()

        def wait_gather(rows, sem):
            pltpu.make_async_copy(
                table_sp.at[idx_v.at[pl.ds(0, CH)]], rows, sem).wait()

        start_gather(0, rows0, sem_g0)

        def body(j, carry):
            i0 = 2 * j
            i1 = 2 * j + 1
            start_gather(i1, rows1, sem_g1)
            wait_gather(rows0, sem_g0)
            pltpu.sync_copy(rows0, out_hbm.at[pl.ds(base + i0 * CH, CH)])

            @pl.when(i1 + 1 < n_ch)
            def _():
                start_gather(i1 + 1, rows0, sem_g0)

            wait_gather(rows1, sem_g1)
            pltpu.sync_copy(rows1, out_hbm.at[pl.ds(base + i1 * CH, CH)])
            return carry

        lax.fori_loop(0, n_ch // 2, body, 0)

    out = gather_kernel(pos_enc_1D, idx_flat)
    return out.reshape(B, S, D)


# speedup vs baseline: 15.9651x; 1.0307x over previous
import functools

import jax
import jax.numpy as jnp
from jax import lax
from jax.experimental import pallas as pl
from jax.experimental.pallas import tpu as pltpu
from jax.experimental.pallas import tpu_sc as plsc

D = 128
NC = 2
NS = 16
NW = NC * NS


def kernel(pos_enc_1D, pos):
    B, S = pos.shape
    V = pos_enc_1D.shape[0]
    N = B * S
    per_w = N // NW
    CH = 256
    n_ch = per_w // CH

    idx_flat = pos.reshape(N)
    mesh = plsc.VectorSubcoreMesh(core_axis_name="c", subcore_axis_name="s")

    @functools.partial(
        pl.kernel,
        mesh=mesh,
        out_type=jax.ShapeDtypeStruct((N, D), jnp.float32),
        scratch_types=[
            pltpu.VMEM((per_w,), jnp.int32),
            pltpu.VMEM((CH, D), jnp.float32),
            pltpu.VMEM((CH, D), jnp.float32),
            pltpu.VMEM((CH, D), jnp.float32),
            pltpu.VMEM_SHARED((V, D), jnp.float32),
            pltpu.SemaphoreType.DMA,
            pltpu.SemaphoreType.DMA,
            pltpu.SemaphoreType.DMA,
            pltpu.SemaphoreType.DMA,
        ],
    )
    def gather_kernel(table_hbm, idx_hbm, out_hbm, idx_v, rows0, rows1,
                      rows2, table_sp, sem_i, sem_g0, sem_g1, sem_g2):
        cid = lax.axis_index("c")
        sid = lax.axis_index("s")
        wid = sid * NC + cid
        base = wid * per_w

        idx_cp = pltpu.make_async_copy(
            idx_hbm.at[pl.ds(base, per_w)], idx_v, sem_i)
        idx_cp.start()

        @pl.when(sid == 0)
        def _():
            pltpu.sync_copy(table_hbm, table_sp)

        plsc.subcore_barrier()
        idx_cp.wait()

        def start_gather(i, rows, sem):
            pltpu.make_async_copy(
                table_sp.at[idx_v.at[pl.ds(i * CH, CH)]], rows, sem).start()

        def wait_gather(rows, sem):
            pltpu.make_async_copy(
                table_sp.at[idx_v.at[pl.ds(0, CH)]], rows, sem).wait()

        start_gather(0, rows0, sem_g0)
        start_gather(1, rows1, sem_g1)

        def body(j, carry):
            i0 = 3 * j
            bufs = ((rows0, sem_g0), (rows1, sem_g1), (rows2, sem_g2))
            for k in range(3):
                rows, sem = bufs[k]
                nxt = i0 + k + 2

                @pl.when(nxt < n_ch)
                def _():
                    nrows, nsem = bufs[(k + 2) % 3]
                    start_gather(nxt, nrows, nsem)

                wait_gather(rows, sem)
                pltpu.sync_copy(
                    rows, out_hbm.at[pl.ds(base + (i0 + k) * CH, CH)])
            return carry

        lax.fori_loop(0, n_ch // 3, body, 0)

        bufs = ((rows0, sem_g0), (rows1, sem_g1), (rows2, sem_g2))
        for i in range(3 * (n_ch // 3), n_ch):
            rows, sem = bufs[i % 3]
            wait_gather(rows, sem)
            pltpu.sync_copy(rows, out_hbm.at[pl.ds(base + i * CH, CH)])

    out = gather_kernel(pos_enc_1D, idx_flat)
    return out.reshape(B, S, D)
